# baseline (speedup 1.0000x reference)
import functools

import jax
import jax.numpy as jnp
from jax.experimental import pallas as pl
from jax.experimental.pallas import tpu as pltpu

N_LAYERS_ = 10
BATCH_ = 32
CHANNELS_ = 1024
L_CACHE_ = 20
LAYER_IDX_ = 0


def _conv_kernel(cp_ref, cache_ref, bx_ref, w_ref, out_ref):
    cp = jnp.clip(cp_ref[0], 0, L_CACHE_ - 1)
    a = cache_ref[0]
    w = w_ref[...]
    bx = bx_ref[0]
    rolled = jnp.concatenate([a[:, 1:], a[:, :1]], axis=1)
    lane = jax.lax.broadcasted_iota(jnp.int32, (CHANNELS_, L_CACHE_), 1)
    state = jnp.where(lane == cp, bx, rolled)
    out_ref[0] = jnp.sum(state * w, axis=1, keepdims=True)


def kernel(Bx, cache_position, seq_len, conv_cache, conv_weight):
    del seq_len
    cache0 = conv_cache[LAYER_IDX_]
    grid_spec = pltpu.PrefetchScalarGridSpec(
        num_scalar_prefetch=1,
        grid=(BATCH_,),
        in_specs=[
            pl.BlockSpec((1, CHANNELS_, L_CACHE_), lambda b, cp: (b, 0, 0)),
            pl.BlockSpec((1, CHANNELS_, 1), lambda b, cp: (b, 0, 0)),
            pl.BlockSpec((CHANNELS_, L_CACHE_), lambda b, cp: (0, 0)),
        ],
        out_specs=pl.BlockSpec((1, CHANNELS_, 1), lambda b, cp: (b, 0, 0)),
    )
    out = pl.pallas_call(
        _conv_kernel,
        grid_spec=grid_spec,
        out_shape=jax.ShapeDtypeStruct((BATCH_, CHANNELS_, 1), jnp.float32),
    )(cache_position, cache0, Bx, conv_weight)
    return out


# speedup vs baseline: 1.6565x; 1.6565x over previous
import jax
import jax.numpy as jnp
import numpy as np
from jax.experimental import pallas as pl
from jax.experimental.pallas import tpu as pltpu

BATCH_ = 32
CHANNELS_ = 1024
L_CACHE_ = 20
LAYER_IDX_ = 0
CB_ = 128
GROUP_ = CB_ * L_CACHE_
NBLK_ = CHANNELS_ // CB_

_j = np.arange(GROUP_)[:, None]
_c = np.arange(CB_)[None, :]
_M0 = (_j // L_CACHE_ == _c).astype(np.float32)
_LPAT = (np.arange(GROUP_) % L_CACHE_).astype(np.int32)[None, :]


def _conv_kernel(cp_ref, a_ref, wr_ref, lpat_ref, m0_ref, bx_ref, out_ref):
    cp = jnp.clip(cp_ref[0], 0, L_CACHE_ - 1)
    cpp = jax.lax.rem(cp + 1, L_CACHE_)
    wr = wr_ref[...]
    lpat = lpat_ref[...]
    keep = (lpat != cpp).astype(jnp.float32)
    sel = (lpat == cpp).astype(jnp.float32)
    w_eff = wr * keep
    w_sel = wr * sel
    p = a_ref[...] * w_eff
    m0 = m0_ref[...]
    red = jnp.dot(p, m0, preferred_element_type=jnp.float32)
    wcp = jnp.dot(w_sel, m0, preferred_element_type=jnp.float32)
    out_ref[...] = red + bx_ref[...] * wcp


def kernel(Bx, cache_position, seq_len, conv_cache, conv_weight):
    del seq_len
    a2 = conv_cache[LAYER_IDX_].reshape(BATCH_, CHANNELS_ * L_CACHE_)
    wr = jnp.roll(conv_weight, 1, axis=1).reshape(1, CHANNELS_ * L_CACHE_)
    bx2 = Bx.reshape(BATCH_, CHANNELS_)
    lpat = jnp.asarray(_LPAT)
    m0 = jnp.asarray(_M0)
    grid_spec = pltpu.PrefetchScalarGridSpec(
        num_scalar_prefetch=1,
        grid=(NBLK_,),
        in_specs=[
            pl.BlockSpec((BATCH_, GROUP_), lambda i, cp: (0, i)),
            pl.BlockSpec((1, GROUP_), lambda i, cp: (0, i)),
            pl.BlockSpec((1, GROUP_), lambda i, cp: (0, 0)),
            pl.BlockSpec((GROUP_, CB_), lambda i, cp: (0, 0)),
            pl.BlockSpec((BATCH_, CB_), lambda i, cp: (0, i)),
        ],
        out_specs=pl.BlockSpec((BATCH_, CB_), lambda i, cp: (0, i)),
    )
    out = pl.pallas_call(
        _conv_kernel,
        grid_spec=grid_spec,
        out_shape=jax.ShapeDtypeStruct((BATCH_, CHANNELS_), jnp.float32),
    )(cache_position, a2, wr, lpat, m0, bx2)
    return out[..., None]


# speedup vs baseline: 1.7509x; 1.0570x over previous
import jax
import jax.numpy as jnp
import numpy as np
from jax.experimental import pallas as pl
from jax.experimental.pallas import tpu as pltpu

BATCH_ = 32
CHANNELS_ = 1024
L_CACHE_ = 20
LAYER_IDX_ = 0
CB_ = 128
GROUP_ = CB_ * L_CACHE_
NBLK_ = CHANNELS_ // CB_

_j = np.arange(GROUP_)[:, None]
_c = np.arange(CB_)[None, :]
_M0 = (_j // L_CACHE_ == _c).astype(np.float32)
_LPAT = (np.arange(CHANNELS_ * L_CACHE_) % L_CACHE_).astype(np.int32)[None, :]


def _conv_kernel(cp_ref, a_ref, wr_ref, lpat_ref, m0_ref, bx_ref, out_ref):
    cp = jnp.clip(cp_ref[0], 0, L_CACHE_ - 1)
    cpp = jax.lax.rem(cp + 1, L_CACHE_)
    wr = wr_ref[...]
    lpat = lpat_ref[...]
    keep = (lpat != cpp).astype(jnp.float32)
    sel = (lpat == cpp).astype(jnp.float32)
    w_eff = wr * keep
    w_sel = wr * sel
    m0 = m0_ref[...]
    for cb in range(NBLK_):
        sl = slice(cb * GROUP_, (cb + 1) * GROUP_)
        osl = slice(cb * CB_, (cb + 1) * CB_)
        p = a_ref[:, sl] * w_eff[:, sl]
        red = jnp.dot(p, m0, preferred_element_type=jnp.float32)
        wcp = jnp.dot(w_sel[:, sl], m0, preferred_element_type=jnp.float32)
        out_ref[:, osl] = red + bx_ref[:, osl] * wcp


def kernel(Bx, cache_position, seq_len, conv_cache, conv_weight):
    del seq_len
    a2 = conv_cache[LAYER_IDX_].reshape(BATCH_, CHANNELS_ * L_CACHE_)
    wr = jnp.roll(conv_weight, 1, axis=1).reshape(1, CHANNELS_ * L_CACHE_)
    bx2 = Bx.reshape(BATCH_, CHANNELS_)
    lpat = jnp.asarray(_LPAT)
    m0 = jnp.asarray(_M0)
    grid_spec = pltpu.PrefetchScalarGridSpec(
        num_scalar_prefetch=1,
        grid=(1,),
        in_specs=[
            pl.BlockSpec((BATCH_, CHANNELS_ * L_CACHE_), lambda i, cp: (0, 0)),
            pl.BlockSpec((1, CHANNELS_ * L_CACHE_), lambda i, cp: (0, 0)),
            pl.BlockSpec((1, CHANNELS_ * L_CACHE_), lambda i, cp: (0, 0)),
            pl.BlockSpec((GROUP_, CB_), lambda i, cp: (0, 0)),
            pl.BlockSpec((BATCH_, CHANNELS_), lambda i, cp: (0, 0)),
        ],
        out_specs=pl.BlockSpec((BATCH_, CHANNELS_), lambda i, cp: (0, 0)),
    )
    out = pl.pallas_call(
        _conv_kernel,
        grid_spec=grid_spec,
        out_shape=jax.ShapeDtypeStruct((BATCH_, CHANNELS_), jnp.float32),
    )(cache_position, a2, wr, lpat, m0, bx2)
    return out[..., None]
